# Initial kernel scaffold; baseline (speedup 1.0000x reference)
import jax
import jax.numpy as jnp
from jax.experimental import pallas as pl


def kernel(hidden_states, lm_head_weight, W_centroids, token_ordering):
    raise NotImplementedError("write your pallas kernel here")


# speedup vs baseline: 3.1659x; 3.1659x over previous
import functools

import jax
import jax.numpy as jnp
from jax import lax
from jax.experimental import pallas as pl
from jax.experimental.pallas import tpu as pltpu
from jax.experimental.pallas import tpu_sc as plsc

_B, _S, _H = 8, 4, 1024
_VOCAB = 131072
_NCENT = 128
_VPC = _VOCAB // _NCENT
_T = _B * _S

_VB_LG = 512
_NBLK_LG = _VOCAB // _VB_LG
_VB_AP = 2048
_NBLK_AP = _VOCAB // _VB_AP


def _router_body(hid_ref, wc_ref, top2_ref):
    h = hid_ref[...]
    wc = wc_ref[...]
    logits = lax.dot_general(
        h, wc, (((1,), (1,)), ((), ())),
        preferred_element_type=jnp.float32,
        precision=lax.Precision.HIGHEST,
    )
    col = lax.broadcasted_iota(jnp.int32, logits.shape, 1)
    m1 = jnp.max(logits, axis=1, keepdims=True)
    i1 = jnp.min(jnp.where(logits == m1, col, _NCENT), axis=1, keepdims=True)
    masked = jnp.where(col == i1, -jnp.inf, logits)
    m2 = jnp.max(masked, axis=1, keepdims=True)
    i2 = jnp.min(jnp.where(masked == m2, col, _NCENT), axis=1, keepdims=True)
    top2_ref[...] = jnp.concatenate([i1, i2], axis=1)


def _router(hid, wc):
    return pl.pallas_call(
        _router_body,
        out_shape=jax.ShapeDtypeStruct((_T, 2), jnp.int32),
    )(hid, wc)


_ROWS_TOTAL = _VOCAB // 128
_ROWS_PER_W = _ROWS_TOTAL // 32


def _invperm_body(tord_hbm, out_hbm, idx_v, vals_v, sem):
    wid = lax.axis_index("s") * 2 + lax.axis_index("c")
    base_row = wid * _ROWS_PER_W
    pltpu.sync_copy(tord_hbm.at[pl.ds(base_row, _ROWS_PER_W)], idx_v)
    for j in range(_ROWS_PER_W):
        val = (base_row + j) // 8
        vec = jnp.full((16,), val, jnp.int32)
        for q in range(8):
            vals_v[j, pl.ds(q * 16, 16)] = vec
    for g in range(0, _ROWS_PER_W, 8):
        handles = [
            pltpu.async_copy(vals_v.at[j], out_hbm.at[idx_v.at[j]], sem)
            for j in range(g, g + 8)
        ]
        for h in handles:
            h.wait()


def _invperm(tord2d):
    mesh = plsc.VectorSubcoreMesh(core_axis_name="c", subcore_axis_name="s")
    fn = functools.partial(
        pl.kernel,
        out_type=jax.ShapeDtypeStruct((_VOCAB,), jnp.int32),
        mesh=mesh,
        scratch_types=[
            pltpu.VMEM((_ROWS_PER_W, 128), jnp.int32),
            pltpu.VMEM((_ROWS_PER_W, 128), jnp.int32),
            pltpu.SemaphoreType.DMA,
        ],
    )(_invperm_body)
    return fn(tord2d)


def _logits_body(hid_ref, w_ref, top2_ref, clu_ref, out_ref, gmin_ref, acc_ref):
    i = pl.program_id(0)
    logits = lax.dot_general(
        hid_ref[...], w_ref[...], (((1,), (1,)), ((), ())),
        preferred_element_type=jnp.float32,
    )
    out_ref[...] = logits
    clu = clu_ref[0]
    mask = (clu == top2_ref[:, 0:1]) | (clu == top2_ref[:, 1:2])
    bmin = jnp.min(jnp.where(mask, logits, jnp.inf))
    prev = jnp.where(i == 0, jnp.inf, acc_ref[0, 0])
    acc = jnp.minimum(prev, bmin)
    acc_ref[0, 0] = acc
    gmin_ref[0, 0] = acc


def _logits_call(hid, w, top2, clu3):
    return pl.pallas_call(
        _logits_body,
        grid=(_NBLK_LG,),
        in_specs=[
            pl.BlockSpec((_T, _H), lambda i: (0, 0)),
            pl.BlockSpec((_VB_LG, _H), lambda i: (i, 0)),
            pl.BlockSpec((_T, 2), lambda i: (0, 0)),
            pl.BlockSpec((1, 1, _VB_LG), lambda i: (i, 0, 0)),
        ],
        out_specs=[
            pl.BlockSpec((_T, _VB_LG), lambda i: (0, i)),
            pl.BlockSpec(memory_space=pltpu.SMEM),
        ],
        out_shape=[
            jax.ShapeDtypeStruct((_T, _VOCAB), jnp.float32),
            jax.ShapeDtypeStruct((1, 1), jnp.float32),
        ],
        scratch_shapes=[pltpu.SMEM((1, 1), jnp.float32)],
    )(hid, w, top2, clu3)


def _apply_body(lg_ref, top2_ref, clu_ref, gmin_ref, out_ref):
    clu = clu_ref[0]
    mask = (clu == top2_ref[:, 0:1]) | (clu == top2_ref[:, 1:2])
    mv = gmin_ref[0, 0] - 1.0
    out_ref[...] = jnp.where(mask, lg_ref[...], mv)


def _apply_call(logits, top2, clu3, gmin):
    return pl.pallas_call(
        _apply_body,
        grid=(_NBLK_AP,),
        in_specs=[
            pl.BlockSpec((_T, _VB_AP), lambda i: (0, i)),
            pl.BlockSpec((_T, 2), lambda i: (0, 0)),
            pl.BlockSpec((1, 1, _VB_AP), lambda i: (i, 0, 0)),
            pl.BlockSpec(memory_space=pltpu.SMEM),
        ],
        out_specs=pl.BlockSpec((_T, _VB_AP), lambda i: (0, i)),
        out_shape=jax.ShapeDtypeStruct((_T, _VOCAB), jnp.float32),
    )(logits, top2, clu3, gmin)


def kernel(hidden_states, lm_head_weight, W_centroids, token_ordering):
    hid = hidden_states.reshape(_T, _H)
    top2 = _router(hid, W_centroids)
    tord2d = token_ordering.astype(jnp.int32).reshape(_ROWS_TOTAL, 128)
    cluster_of = _invperm(tord2d)
    clu_lg = cluster_of.reshape(_NBLK_LG, 1, _VB_LG)
    clu_ap = cluster_of.reshape(_NBLK_AP, 1, _VB_AP)
    logits, gmin = _logits_call(hid, lm_head_weight, top2, clu_lg)
    out = _apply_call(logits, top2, clu_ap, gmin)
    return out.reshape(_B, _S, _VOCAB)


# speedup vs baseline: 4.3394x; 1.3707x over previous
import functools

import jax
import jax.numpy as jnp
from jax import lax
from jax.experimental import pallas as pl
from jax.experimental.pallas import tpu as pltpu
from jax.experimental.pallas import tpu_sc as plsc

_B, _S, _H = 8, 4, 1024
_VOCAB = 131072
_NCENT = 128
_VPC = _VOCAB // _NCENT
_T = _B * _S

_VB_LG = 1024
_NBLK_LG = _VOCAB // _VB_LG
_VB_AP = 2048
_NBLK_AP = _VOCAB // _VB_AP


def _router_body(hid_ref, wc_ref, top2_ref):
    h = hid_ref[...]
    wc = wc_ref[...]
    logits = lax.dot_general(
        h, wc, (((1,), (1,)), ((), ())),
        preferred_element_type=jnp.float32,
        precision=lax.Precision.HIGHEST,
    )
    col = lax.broadcasted_iota(jnp.int32, logits.shape, 1)
    m1 = jnp.max(logits, axis=1, keepdims=True)
    i1 = jnp.min(jnp.where(logits == m1, col, _NCENT), axis=1, keepdims=True)
    masked = jnp.where(col == i1, -jnp.inf, logits)
    m2 = jnp.max(masked, axis=1, keepdims=True)
    i2 = jnp.min(jnp.where(masked == m2, col, _NCENT), axis=1, keepdims=True)
    top2_ref[...] = jnp.concatenate([i1, i2], axis=1)


def _router(hid, wc):
    return pl.pallas_call(
        _router_body,
        out_shape=jax.ShapeDtypeStruct((_T, 2), jnp.int32),
    )(hid, wc)


_ROWS_TOTAL = _VOCAB // 128
_ROWS_PER_W = _ROWS_TOTAL // 32


def _invperm_body(tord_hbm, out_hbm, idx_v, vals_v, sem):
    wid = lax.axis_index("s") * 2 + lax.axis_index("c")
    base_row = wid * _ROWS_PER_W
    pltpu.sync_copy(tord_hbm.at[pl.ds(base_row, _ROWS_PER_W)], idx_v)
    for j in range(_ROWS_PER_W):
        val = (base_row + j) // 8
        vec = jnp.full((16,), val, jnp.int32)
        for q in range(8):
            vals_v[j, pl.ds(q * 16, 16)] = vec
    for g in range(0, _ROWS_PER_W, 8):
        handles = [
            pltpu.async_copy(vals_v.at[j], out_hbm.at[idx_v.at[j]], sem)
            for j in range(g, g + 8)
        ]
        for h in handles:
            h.wait()


def _invperm(tord2d):
    mesh = plsc.VectorSubcoreMesh(core_axis_name="c", subcore_axis_name="s")
    fn = functools.partial(
        pl.kernel,
        out_type=jax.ShapeDtypeStruct((_VOCAB,), jnp.int32),
        mesh=mesh,
        scratch_types=[
            pltpu.VMEM((_ROWS_PER_W, 128), jnp.int32),
            pltpu.VMEM((_ROWS_PER_W, 128), jnp.int32),
            pltpu.SemaphoreType.DMA,
        ],
    )(_invperm_body)
    return fn(tord2d)


def _logits_body(hid_ref, w_ref, out_ref):
    out_ref[...] = lax.dot_general(
        hid_ref[...], w_ref[...], (((1,), (1,)), ((), ())),
        preferred_element_type=jnp.float32,
    )


def _logits_call(hid, w):
    return pl.pallas_call(
        _logits_body,
        grid=(_NBLK_LG,),
        in_specs=[
            pl.BlockSpec((_T, _H), lambda i: (0, 0)),
            pl.BlockSpec((_VB_LG, _H), lambda i: (i, 0)),
        ],
        out_specs=pl.BlockSpec((_T, _VB_LG), lambda i: (0, i)),
        out_shape=jax.ShapeDtypeStruct((_T, _VOCAB), jnp.float32),
    )(hid, w)


def _minpass_body(lg_ref, top2_ref, clu_ref, gmin_ref, acc_ref):
    i = pl.program_id(0)
    clu = clu_ref[0]
    mask = (clu == top2_ref[:, 0:1]) | (clu == top2_ref[:, 1:2])
    bmin = jnp.min(jnp.where(mask, lg_ref[...], jnp.inf))
    prev = jnp.where(i == 0, jnp.inf, acc_ref[0, 0])
    acc = jnp.minimum(prev, bmin)
    acc_ref[0, 0] = acc
    gmin_ref[0, 0] = acc


def _minpass_call(logits, top2, clu3):
    return pl.pallas_call(
        _minpass_body,
        grid=(_NBLK_AP,),
        in_specs=[
            pl.BlockSpec((_T, _VB_AP), lambda i: (0, i)),
            pl.BlockSpec((_T, 2), lambda i: (0, 0)),
            pl.BlockSpec((1, 1, _VB_AP), lambda i: (i, 0, 0)),
        ],
        out_specs=pl.BlockSpec(memory_space=pltpu.SMEM),
        out_shape=jax.ShapeDtypeStruct((1, 1), jnp.float32),
        scratch_shapes=[pltpu.SMEM((1, 1), jnp.float32)],
    )(logits, top2, clu3)


def _apply_body(lg_ref, top2_ref, clu_ref, gmin_ref, out_ref):
    clu = clu_ref[0]
    mask = (clu == top2_ref[:, 0:1]) | (clu == top2_ref[:, 1:2])
    mv = gmin_ref[0, 0] - 1.0
    out_ref[...] = jnp.where(mask, lg_ref[...], mv)


def _apply_call(logits, top2, clu3, gmin):
    return pl.pallas_call(
        _apply_body,
        grid=(_NBLK_AP,),
        in_specs=[
            pl.BlockSpec((_T, _VB_AP), lambda i: (0, i)),
            pl.BlockSpec((_T, 2), lambda i: (0, 0)),
            pl.BlockSpec((1, 1, _VB_AP), lambda i: (i, 0, 0)),
            pl.BlockSpec(memory_space=pltpu.SMEM),
        ],
        out_specs=pl.BlockSpec((_T, _VB_AP), lambda i: (0, i)),
        out_shape=jax.ShapeDtypeStruct((_T, _VOCAB), jnp.float32),
    )(logits, top2, clu3, gmin)


def kernel(hidden_states, lm_head_weight, W_centroids, token_ordering):
    hid = hidden_states.reshape(_T, _H)
    top2 = _router(hid, W_centroids)
    tord2d = token_ordering.astype(jnp.int32).reshape(_ROWS_TOTAL, 128)
    cluster_of = _invperm(tord2d)
    clu_ap = cluster_of.reshape(_NBLK_AP, 1, _VB_AP)
    logits = _logits_call(hid, lm_head_weight)
    gmin = _minpass_call(logits, top2, clu_ap)
    out = _apply_call(logits, top2, clu_ap, gmin)
    return out.reshape(_B, _S, _VOCAB)
